# Initial kernel scaffold; baseline (speedup 1.0000x reference)
import jax
import jax.numpy as jnp
from jax.experimental import pallas as pl


def kernel(x, edge_attr, edge_index, batch, W_node, b_node, W_edge, b_edge, W1, b1, W2, b2):
    raise NotImplementedError("write your pallas kernel here")


# speedup vs baseline: 2.2355x; 2.2355x over previous
import functools

import jax
import jax.numpy as jnp
from jax import lax
from jax.experimental import pallas as pl
from jax.experimental.pallas import tpu as pltpu
from jax.experimental.pallas import tpu_sc as plsc

N = 10000
E = 320000
D_IN = 128
D_EDGE = 16
HID = 128
DEPTH = 3
NUM_GRAPHS = 64

NC = 2
NS = 16
NW = NC * NS
BLK = 128
RW = -(-E // (NW * BLK))
EPAD = NW * BLK * RW
NPAD = 10240
STRIPE = NPAD // NS


def _proj_h_body(x_ref, w_ref, b_ref, o_ref):
    o_ref[...] = jnp.maximum(
        jnp.dot(x_ref[...], w_ref[...], preferred_element_type=jnp.float32)
        + b_ref[...], 0.0)


def _proj_h(x, w, b):
    blk = 1000
    return pl.pallas_call(
        _proj_h_body,
        grid=(N // blk,),
        in_specs=[
            pl.BlockSpec((blk, D_IN), lambda i: (i, 0)),
            pl.BlockSpec((D_IN, HID), lambda i: (0, 0)),
            pl.BlockSpec((1, HID), lambda i: (0, 0)),
        ],
        out_specs=pl.BlockSpec((blk, HID), lambda i: (i, 0)),
        out_shape=jax.ShapeDtypeStruct((N, HID), jnp.float32),
    )(x, w, b)


def _proj_e_body(ea_ref, w_ref, b_ref, o_ref):
    o_ref[...] = (
        jnp.dot(ea_ref[...], w_ref[...], preferred_element_type=jnp.float32)
        + b_ref[...])


def _proj_e(edge_attr, w, b):
    blk = 512
    nblk_real = E // blk
    return pl.pallas_call(
        _proj_e_body,
        grid=(EPAD // blk,),
        in_specs=[
            pl.BlockSpec((blk, D_EDGE),
                         lambda i: (jnp.minimum(i, nblk_real - 1), 0)),
            pl.BlockSpec((D_EDGE, HID), lambda i: (0, 0)),
            pl.BlockSpec((1, HID), lambda i: (0, 0)),
        ],
        out_specs=pl.BlockSpec((blk, HID), lambda i: (i, 0)),
        out_shape=jax.ShapeDtypeStruct((EPAD, HID), jnp.float32),
    )(edge_attr, w, b)


def _mlp_body(h_ref, a0_ref, a1_ref, w1_ref, b1_ref, w2_ref, b2_ref, o_ref,
              *, final_relu):
    z = h_ref[...] + a0_ref[0] + a1_ref[0]
    z = jnp.maximum(
        jnp.dot(z, w1_ref[...], preferred_element_type=jnp.float32)
        + b1_ref[...], 0.0)
    z = jnp.dot(z, w2_ref[...], preferred_element_type=jnp.float32) + b2_ref[...]
    if final_relu:
        z = jnp.maximum(z, 0.0)
    o_ref[...] = z


def _mlp(h, agg, w1, b1, w2, b2, final_relu):
    blk = 1000
    return pl.pallas_call(
        functools.partial(_mlp_body, final_relu=final_relu),
        grid=(N // blk,),
        in_specs=[
            pl.BlockSpec((blk, HID), lambda i: (i, 0)),
            pl.BlockSpec((1, blk, HID), lambda i: (0, i, 0)),
            pl.BlockSpec((1, blk, HID), lambda i: (1, i, 0)),
            pl.BlockSpec((HID, HID), lambda i: (0, 0)),
            pl.BlockSpec((1, HID), lambda i: (0, 0)),
            pl.BlockSpec((HID, HID), lambda i: (0, 0)),
            pl.BlockSpec((1, HID), lambda i: (0, 0)),
        ],
        out_specs=pl.BlockSpec((blk, HID), lambda i: (i, 0)),
        out_shape=jax.ShapeDtypeStruct((N, HID), jnp.float32),
    )(h, agg, agg, w1, b1, w2, b2)


def _mlp_pool_body(h_ref, a0_ref, a1_ref, w1_ref, b1_ref, w2_ref, b2_ref,
                   batch_ref, o_ref):
    i = pl.program_id(0)
    z = h_ref[...] + a0_ref[0] + a1_ref[0]
    z = jnp.maximum(
        jnp.dot(z, w1_ref[...], preferred_element_type=jnp.float32)
        + b1_ref[...], 0.0)
    z = jnp.dot(z, w2_ref[...], preferred_element_type=jnp.float32) + b2_ref[...]
    gids = lax.broadcasted_iota(jnp.int32, (z.shape[0], NUM_GRAPHS), 1)
    onehot = (batch_ref[...] == gids).astype(jnp.float32)
    part = lax.dot_general(onehot, z, (((0,), (0,)), ((), ())),
                           preferred_element_type=jnp.float32)

    @pl.when(i == 0)
    def _():
        o_ref[...] = jnp.zeros_like(o_ref)

    o_ref[...] += part


def _mlp_pool(h, agg, w1, b1, w2, b2, batch2d):
    blk = 1000
    return pl.pallas_call(
        _mlp_pool_body,
        grid=(N // blk,),
        in_specs=[
            pl.BlockSpec((blk, HID), lambda i: (i, 0)),
            pl.BlockSpec((1, blk, HID), lambda i: (0, i, 0)),
            pl.BlockSpec((1, blk, HID), lambda i: (1, i, 0)),
            pl.BlockSpec((HID, HID), lambda i: (0, 0)),
            pl.BlockSpec((1, HID), lambda i: (0, 0)),
            pl.BlockSpec((HID, HID), lambda i: (0, 0)),
            pl.BlockSpec((1, HID), lambda i: (0, 0)),
            pl.BlockSpec((blk, 1), lambda i: (i, 0)),
        ],
        out_specs=pl.BlockSpec((NUM_GRAPHS, HID), lambda i: (0, 0)),
        out_shape=jax.ShapeDtypeStruct((NUM_GRAPHS, HID), jnp.float32),
    )(h, agg, agg, w1, b1, w2, b2, batch2d)


def _sc_agg_body(h_hbm, e_hbm, src_hbm, dst_hbm, out_hbm,
                 srcv, dstv, hrows, erows, aggsh, sem):
    c = lax.axis_index("c")
    s = lax.axis_index("s")
    w = s * NC + c

    def zrow(i, _):
        for j in range(HID // 16):
            erows[i, pl.ds(j * 16, 16)] = jnp.zeros((16,), jnp.float32)
        return 0
    lax.fori_loop(0, BLK, zrow, 0)
    for k in range(STRIPE // BLK):
        pltpu.sync_copy(erows, aggsh.at[pl.ds(s * STRIPE + k * BLK, BLK)])
    plsc.subcore_barrier()

    def edge_chunk(r, _):
        base = (w * RW + r) * BLK
        pltpu.sync_copy(src_hbm.at[pl.ds(base, BLK)], srcv)
        pltpu.sync_copy(dst_hbm.at[pl.ds(base, BLK)], dstv)
        pltpu.async_copy(h_hbm.at[srcv], hrows, sem).wait()
        pltpu.sync_copy(e_hbm.at[pl.ds(base, BLK)], erows)

        def msg_row(i, _):
            for j in range(HID // 16):
                sl = pl.ds(j * 16, 16)
                erows[i, sl] = jnp.maximum(hrows[i, sl] + erows[i, sl], 0.0)
            return 0
        lax.fori_loop(0, BLK, msg_row, 0)
        pltpu.sync_copy(erows, aggsh.at[dstv], add=True)
        return 0

    lax.fori_loop(0, RW, edge_chunk, 0)
    plsc.subcore_barrier()
    pltpu.sync_copy(aggsh.at[pl.ds(s * STRIPE, STRIPE)],
                    out_hbm.at[c, pl.ds(s * STRIPE, STRIPE)])


_sc_agg = functools.partial(
    pl.kernel,
    out_type=jax.ShapeDtypeStruct((NC, NPAD, HID), jnp.float32),
    mesh=plsc.VectorSubcoreMesh(core_axis_name="c", subcore_axis_name="s"),
    scratch_types=[
        pltpu.VMEM((BLK,), jnp.int32),
        pltpu.VMEM((BLK,), jnp.int32),
        pltpu.VMEM((BLK, HID), jnp.float32),
        pltpu.VMEM((BLK, HID), jnp.float32),
        pltpu.VMEM_SHARED((NPAD, HID), jnp.float32),
        pltpu.SemaphoreType.DMA,
    ],
)(_sc_agg_body)


def kernel(x, edge_attr, edge_index, batch,
           W_node, b_node, W_edge, b_edge, W1, b1, W2, b2):
    src = edge_index[0]
    dst = edge_index[1]
    pad = EPAD - E
    src_p = jnp.concatenate([src, jnp.zeros((pad,), jnp.int32)])
    dst_p = jnp.concatenate([dst, jnp.full((pad,), N, jnp.int32)])
    batch2d = batch.reshape(N, 1)

    h = _proj_h(x, W_node, b_node.reshape(1, HID))
    e = _proj_e(edge_attr, W_edge, b_edge.reshape(1, HID))

    for i in range(DEPTH):
        agg = _sc_agg(h, e, src_p, dst_p)
        if i < DEPTH - 1:
            h = _mlp(h, agg, W1[i], b1[i].reshape(1, HID),
                     W2[i], b2[i].reshape(1, HID), final_relu=True)
        else:
            out = _mlp_pool(h, agg, W1[i], b1[i].reshape(1, HID),
                            W2[i], b2[i].reshape(1, HID), batch2d)
    return out


# speedup vs baseline: 2.4118x; 1.0788x over previous
import functools

import jax
import jax.numpy as jnp
from jax import lax
from jax.experimental import pallas as pl
from jax.experimental.pallas import tpu as pltpu
from jax.experimental.pallas import tpu_sc as plsc

N = 10000
E = 320000
D_IN = 128
D_EDGE = 16
HID = 128
DEPTH = 3
NUM_GRAPHS = 64

NC = 2
NS = 16
NW = NC * NS
BLK = 64
RW = 160
EPAD = NW * BLK * RW
NPAD = 10240
STRIPE = NPAD // NS


def _proj_h_body(x_ref, w_ref, b_ref, o_ref):
    o_ref[...] = jnp.maximum(
        jnp.dot(x_ref[...], w_ref[...], preferred_element_type=jnp.float32)
        + b_ref[...], 0.0)


def _proj_h(x, w, b):
    blk = 1000
    return pl.pallas_call(
        _proj_h_body,
        grid=(N // blk,),
        in_specs=[
            pl.BlockSpec((blk, D_IN), lambda i: (i, 0)),
            pl.BlockSpec((D_IN, HID), lambda i: (0, 0)),
            pl.BlockSpec((1, HID), lambda i: (0, 0)),
        ],
        out_specs=pl.BlockSpec((blk, HID), lambda i: (i, 0)),
        out_shape=jax.ShapeDtypeStruct((N, HID), jnp.float32),
    )(x, w, b)


def _proj_e_body(ea_ref, w_ref, b_ref, o_ref):
    z = (jnp.dot(ea_ref[...], w_ref[...], preferred_element_type=jnp.float32)
         + b_ref[...])
    o_ref[...] = z.reshape(o_ref.shape)


def _proj_e(edge_attr, w, b):
    blk = 512
    nblk_real = E // blk
    return pl.pallas_call(
        _proj_e_body,
        grid=(EPAD // blk,),
        in_specs=[
            pl.BlockSpec((blk, D_EDGE),
                         lambda i: (jnp.minimum(i, nblk_real - 1), 0)),
            pl.BlockSpec((D_EDGE, HID), lambda i: (0, 0)),
            pl.BlockSpec((1, HID), lambda i: (0, 0)),
        ],
        out_specs=pl.BlockSpec((blk // BLK, BLK, HID), lambda i: (i, 0, 0)),
        out_shape=jax.ShapeDtypeStruct((EPAD // BLK, BLK, HID), jnp.float32),
    )(edge_attr, w, b)


def _mlp_core(h, a0, a1, w1_ref, b1_ref, w2_ref, b2_ref):
    z = h + a0 + a1
    z = jnp.maximum(
        jnp.dot(z, w1_ref[...], preferred_element_type=jnp.float32)
        + b1_ref[...], 0.0)
    return jnp.dot(z, w2_ref[...], preferred_element_type=jnp.float32) \
        + b2_ref[...]


def _mlp_body(h_ref, a0_ref, a1_ref, w1_ref, b1_ref, w2_ref, b2_ref, o_ref):
    z = _mlp_core(h_ref[...], a0_ref[0], a1_ref[0],
                  w1_ref, b1_ref, w2_ref, b2_ref)
    o_ref[...] = jnp.maximum(z, 0.0)


def _mlp(h, agg, w1, b1, w2, b2):
    blk = 1000
    return pl.pallas_call(
        _mlp_body,
        grid=(N // blk,),
        in_specs=[
            pl.BlockSpec((blk, HID), lambda i: (i, 0)),
            pl.BlockSpec((1, blk, HID), lambda i: (0, i, 0)),
            pl.BlockSpec((1, blk, HID), lambda i: (1, i, 0)),
            pl.BlockSpec((HID, HID), lambda i: (0, 0)),
            pl.BlockSpec((1, HID), lambda i: (0, 0)),
            pl.BlockSpec((HID, HID), lambda i: (0, 0)),
            pl.BlockSpec((1, HID), lambda i: (0, 0)),
        ],
        out_specs=pl.BlockSpec((blk, HID), lambda i: (i, 0)),
        out_shape=jax.ShapeDtypeStruct((N, HID), jnp.float32),
    )(h, agg, agg, w1, b1, w2, b2)


def _mlp_pool_body(h_ref, a0_ref, a1_ref, w1_ref, b1_ref, w2_ref, b2_ref,
                   batch_ref, o_ref):
    i = pl.program_id(0)
    z = _mlp_core(h_ref[...], a0_ref[0], a1_ref[0],
                  w1_ref, b1_ref, w2_ref, b2_ref)
    gids = lax.broadcasted_iota(jnp.int32, (z.shape[0], NUM_GRAPHS), 1)
    onehot = (batch_ref[...] == gids).astype(jnp.float32)
    part = lax.dot_general(onehot, z, (((0,), (0,)), ((), ())),
                           preferred_element_type=jnp.float32)

    @pl.when(i == 0)
    def _():
        o_ref[...] = jnp.zeros_like(o_ref)

    o_ref[...] += part


def _mlp_pool(h, agg, w1, b1, w2, b2, batch2d):
    blk = 1000
    return pl.pallas_call(
        _mlp_pool_body,
        grid=(N // blk,),
        in_specs=[
            pl.BlockSpec((blk, HID), lambda i: (i, 0)),
            pl.BlockSpec((1, blk, HID), lambda i: (0, i, 0)),
            pl.BlockSpec((1, blk, HID), lambda i: (1, i, 0)),
            pl.BlockSpec((HID, HID), lambda i: (0, 0)),
            pl.BlockSpec((1, HID), lambda i: (0, 0)),
            pl.BlockSpec((HID, HID), lambda i: (0, 0)),
            pl.BlockSpec((1, HID), lambda i: (0, 0)),
            pl.BlockSpec((blk, 1), lambda i: (i, 0)),
        ],
        out_specs=pl.BlockSpec((NUM_GRAPHS, HID), lambda i: (0, 0)),
        out_shape=jax.ShapeDtypeStruct((NUM_GRAPHS, HID), jnp.float32),
    )(h, agg, agg, w1, b1, w2, b2, batch2d)


def _sc_agg_body(h_hbm, e_hbm, src_hbm, dst_hbm, out_hbm,
                 srcv, dstv, hrows, erows, aggsh,
                 si0, si1, sj0, sj1, sj2, sj3, sg0, sg1, se0, se1, ss0, ss1):
    si = (si0, si1)
    sj = (sj0, sj1, sj2, sj3)
    sg = (sg0, sg1)
    se = (se0, se1)
    ss = (ss0, ss1)
    c = lax.axis_index("c")
    s = lax.axis_index("s")
    w = s * NC + c

    def zrow(i, _):
        for j in range(HID // 16):
            erows[0, i, pl.ds(j * 16, 16)] = jnp.zeros((16,), jnp.float32)
        return 0
    lax.fori_loop(0, BLK, zrow, 0)
    for k in range(STRIPE // BLK):
        pltpu.sync_copy(erows.at[0], aggsh.at[pl.ds(s * STRIPE + k * BLK, BLK)])
    plsc.subcore_barrier()

    def issue_src(r, slot):
        pltpu.async_copy(src_hbm.at[w, r], srcv.at[slot], si[slot])

    def wait_src(r, slot):
        pltpu.make_async_copy(src_hbm.at[w, r], srcv.at[slot],
                              si[slot]).wait()

    def issue_dst(r, slot):
        pltpu.async_copy(dst_hbm.at[w, r], dstv.at[slot], sj[slot])

    def wait_dst(r, slot):
        pltpu.make_async_copy(dst_hbm.at[w, r], dstv.at[slot],
                              sj[slot]).wait()

    def issue_gather(r, slot):
        pltpu.async_copy(h_hbm.at[srcv.at[slot]], hrows.at[slot], sg[slot])

    def wait_gather(r, slot):
        pltpu.make_async_copy(h_hbm.at[srcv.at[slot]], hrows.at[slot],
                              sg[slot]).wait()

    def issue_e(r, slot):
        pltpu.async_copy(e_hbm.at[w * RW + r], erows.at[slot], se[slot])

    def wait_e(r, slot):
        pltpu.make_async_copy(e_hbm.at[w * RW + r], erows.at[slot],
                              se[slot]).wait()

    def issue_scatter(r, slot, jslot):
        pltpu.async_copy(erows.at[slot], aggsh.at[dstv.at[jslot]], ss[slot],
                         add=True)

    def wait_scatter(r, slot, jslot):
        pltpu.make_async_copy(erows.at[slot], aggsh.at[dstv.at[jslot]],
                              ss[slot]).wait()

    issue_src(0, 0)
    issue_src(1, 1)
    issue_dst(0, 0)
    issue_dst(1, 1)
    wait_src(0, 0)
    issue_gather(0, 0)
    issue_e(0, 0)

    def outer(it, _):
        r0 = it * 4
        for b in range(4):
            r = r0 + b
            b2 = b % 2
            nb2 = 1 - b2

            @pl.when(r + 1 < RW)
            def _():
                wait_src(r + 1, nb2)
                issue_gather(r + 1, nb2)

            wait_gather(r, b2)
            wait_e(r, b2)

            @pl.when(r + 2 < RW)
            def _():
                issue_src(r + 2, b2)
                issue_dst(r + 2, (b + 2) % 4)

            def msg_row(i, _):
                for j in range(HID // 16):
                    sl = pl.ds(j * 16, 16)
                    erows[b2, i, sl] = jnp.maximum(
                        hrows[b2, i, sl] + erows[b2, i, sl], 0.0)
                return 0
            lax.fori_loop(0, BLK, msg_row, 0)

            wait_dst(r, b)
            issue_scatter(r, b2, b)

            @pl.when(r >= 1)
            def _():
                wait_scatter(r - 1, nb2, (b + 3) % 4)

            @pl.when(r + 1 < RW)
            def _():
                issue_e(r + 1, nb2)
        return 0

    lax.fori_loop(0, RW // 4, outer, 0)
    wait_scatter(RW - 1, (RW - 1) % 2, (RW - 1) % 4)

    plsc.subcore_barrier()
    pltpu.sync_copy(aggsh.at[pl.ds(s * STRIPE, STRIPE)],
                    out_hbm.at[c, pl.ds(s * STRIPE, STRIPE)])


_sc_agg = functools.partial(
    pl.kernel,
    out_type=jax.ShapeDtypeStruct((NC, NPAD, HID), jnp.float32),
    mesh=plsc.VectorSubcoreMesh(core_axis_name="c", subcore_axis_name="s"),
    scratch_types=[
        pltpu.VMEM((2, BLK), jnp.int32),
        pltpu.VMEM((4, BLK), jnp.int32),
        pltpu.VMEM((2, BLK, HID), jnp.float32),
        pltpu.VMEM((2, BLK, HID), jnp.float32),
        pltpu.VMEM_SHARED((NPAD, HID), jnp.float32),
        pltpu.SemaphoreType.DMA,
        pltpu.SemaphoreType.DMA,
        pltpu.SemaphoreType.DMA,
        pltpu.SemaphoreType.DMA,
        pltpu.SemaphoreType.DMA,
        pltpu.SemaphoreType.DMA,
        pltpu.SemaphoreType.DMA,
        pltpu.SemaphoreType.DMA,
        pltpu.SemaphoreType.DMA,
        pltpu.SemaphoreType.DMA,
        pltpu.SemaphoreType.DMA,
        pltpu.SemaphoreType.DMA,
    ],
)(_sc_agg_body)


def kernel(x, edge_attr, edge_index, batch,
           W_node, b_node, W_edge, b_edge, W1, b1, W2, b2):
    src = edge_index[0]
    dst = edge_index[1]
    pad = EPAD - E
    src_p = jnp.concatenate([src, jnp.zeros((pad,), jnp.int32)])
    src_p = src_p.reshape(NW, RW, BLK)
    dst_p = jnp.concatenate([dst, jnp.full((pad,), N, jnp.int32)])
    dst_p = dst_p.reshape(NW, RW, BLK)
    batch2d = batch.reshape(N, 1)

    h = _proj_h(x, W_node, b_node.reshape(1, HID))
    e = _proj_e(edge_attr, W_edge, b_edge.reshape(1, HID))

    for i in range(DEPTH):
        agg = _sc_agg(h, e, src_p, dst_p)
        if i < DEPTH - 1:
            h = _mlp(h, agg, W1[i], b1[i].reshape(1, HID),
                     W2[i], b2[i].reshape(1, HID))
        else:
            out = _mlp_pool(h, agg, W1[i], b1[i].reshape(1, HID),
                            W2[i], b2[i].reshape(1, HID), batch2d)
    return out


# speedup vs baseline: 2.8504x; 1.1819x over previous
import functools

import jax
import jax.numpy as jnp
from jax import lax
from jax.experimental import pallas as pl
from jax.experimental.pallas import tpu as pltpu
from jax.experimental.pallas import tpu_sc as plsc

N = 10000
E = 320000
D_IN = 128
D_EDGE = 16
HID = 128
DEPTH = 3
NUM_GRAPHS = 64

NC = 2
NS = 16
NW = NC * NS
BLK = 64
RW = 159
EPAD = NW * BLK * RW
NPAD = 10112
STRIPE = NPAD // NS


def _proj_h_body(x_ref, w_ref, b_ref, o_ref):
    o_ref[...] = jnp.maximum(
        jnp.dot(x_ref[...], w_ref[...], preferred_element_type=jnp.float32)
        + b_ref[...], 0.0)


def _proj_h(x, w, b):
    blk = 1000
    return pl.pallas_call(
        _proj_h_body,
        grid=(N // blk,),
        in_specs=[
            pl.BlockSpec((blk, D_IN), lambda i: (i, 0)),
            pl.BlockSpec((D_IN, HID), lambda i: (0, 0)),
            pl.BlockSpec((1, HID), lambda i: (0, 0)),
        ],
        out_specs=pl.BlockSpec((blk, HID), lambda i: (i, 0)),
        out_shape=jax.ShapeDtypeStruct((N, HID), jnp.float32),
    )(x, w, b)


def _proj_e_body(ea_ref, w_ref, b_ref, o_ref):
    z = (jnp.dot(ea_ref[...], w_ref[...], preferred_element_type=jnp.float32)
         + b_ref[...])
    o_ref[...] = z.reshape(o_ref.shape)


def _proj_e(edge_attr, w, b):
    blk = 512
    nblk_real = E // blk
    return pl.pallas_call(
        _proj_e_body,
        grid=(EPAD // blk,),
        in_specs=[
            pl.BlockSpec((blk, D_EDGE),
                         lambda i: (jnp.minimum(i, nblk_real - 1), 0)),
            pl.BlockSpec((D_EDGE, HID), lambda i: (0, 0)),
            pl.BlockSpec((1, HID), lambda i: (0, 0)),
        ],
        out_specs=pl.BlockSpec((blk // BLK, BLK, HID), lambda i: (i, 0, 0)),
        out_shape=jax.ShapeDtypeStruct((EPAD // BLK, BLK, HID), jnp.float32),
    )(edge_attr, w, b)


def _mlp_core(h, a0, a1, w1_ref, b1_ref, w2_ref, b2_ref):
    z = h + a0 + a1
    z = jnp.maximum(
        jnp.dot(z, w1_ref[...], preferred_element_type=jnp.float32)
        + b1_ref[...], 0.0)
    return jnp.dot(z, w2_ref[...], preferred_element_type=jnp.float32) \
        + b2_ref[...]


def _mlp_body(h_ref, a0_ref, a1_ref, w1_ref, b1_ref, w2_ref, b2_ref, o_ref):
    z = _mlp_core(h_ref[...], a0_ref[0], a1_ref[0],
                  w1_ref, b1_ref, w2_ref, b2_ref)
    o_ref[...] = jnp.maximum(z, 0.0)


def _mlp(h, agg, w1, b1, w2, b2):
    blk = 1000
    return pl.pallas_call(
        _mlp_body,
        grid=(N // blk,),
        in_specs=[
            pl.BlockSpec((blk, HID), lambda i: (i, 0)),
            pl.BlockSpec((1, blk, HID), lambda i: (0, i, 0)),
            pl.BlockSpec((1, blk, HID), lambda i: (1, i, 0)),
            pl.BlockSpec((HID, HID), lambda i: (0, 0)),
            pl.BlockSpec((1, HID), lambda i: (0, 0)),
            pl.BlockSpec((HID, HID), lambda i: (0, 0)),
            pl.BlockSpec((1, HID), lambda i: (0, 0)),
        ],
        out_specs=pl.BlockSpec((blk, HID), lambda i: (i, 0)),
        out_shape=jax.ShapeDtypeStruct((N, HID), jnp.float32),
    )(h, agg, agg, w1, b1, w2, b2)


def _mlp_pool_body(h_ref, a0_ref, a1_ref, w1_ref, b1_ref, w2_ref, b2_ref,
                   batch_ref, o_ref):
    i = pl.program_id(0)
    z = _mlp_core(h_ref[...], a0_ref[0], a1_ref[0],
                  w1_ref, b1_ref, w2_ref, b2_ref)
    gids = lax.broadcasted_iota(jnp.int32, (z.shape[0], NUM_GRAPHS), 1)
    onehot = (batch_ref[...] == gids).astype(jnp.float32)
    part = lax.dot_general(onehot, z, (((0,), (0,)), ((), ())),
                           preferred_element_type=jnp.float32)

    @pl.when(i == 0)
    def _():
        o_ref[...] = jnp.zeros_like(o_ref)

    o_ref[...] += part


def _mlp_pool(h, agg, w1, b1, w2, b2, batch2d):
    blk = 1000
    return pl.pallas_call(
        _mlp_pool_body,
        grid=(N // blk,),
        in_specs=[
            pl.BlockSpec((blk, HID), lambda i: (i, 0)),
            pl.BlockSpec((1, blk, HID), lambda i: (0, i, 0)),
            pl.BlockSpec((1, blk, HID), lambda i: (1, i, 0)),
            pl.BlockSpec((HID, HID), lambda i: (0, 0)),
            pl.BlockSpec((1, HID), lambda i: (0, 0)),
            pl.BlockSpec((HID, HID), lambda i: (0, 0)),
            pl.BlockSpec((1, HID), lambda i: (0, 0)),
            pl.BlockSpec((blk, 1), lambda i: (i, 0)),
        ],
        out_specs=pl.BlockSpec((NUM_GRAPHS, HID), lambda i: (0, 0)),
        out_shape=jax.ShapeDtypeStruct((NUM_GRAPHS, HID), jnp.float32),
    )(h, agg, agg, w1, b1, w2, b2, batch2d)


def _sc_agg_body(h_hbm, e_hbm, src_hbm, dst_hbm, out_hbm,
                 srcv, dstv, hrows, erows, aggsh,
                 si0, si1, si2, sj0, sj1, sj2, sg0, sg1, sg2,
                 se0, se1, se2, ss0, ss1, ss2):
    si = (si0, si1, si2)
    sj = (sj0, sj1, sj2)
    sg = (sg0, sg1, sg2)
    se = (se0, se1, se2)
    ss = (ss0, ss1, ss2)
    c = lax.axis_index("c")
    s = lax.axis_index("s")
    w = s * NC + c

    def zrow(i, _):
        for j in range(HID // 16):
            erows[0, i, pl.ds(j * 16, 16)] = jnp.zeros((16,), jnp.float32)
        return 0
    lax.fori_loop(0, BLK, zrow, 0)
    for k in range(STRIPE // BLK):
        pltpu.sync_copy(erows.at[0], aggsh.at[pl.ds(s * STRIPE + k * BLK, BLK)])
    rem_rows = STRIPE - (STRIPE // BLK) * BLK
    if rem_rows:
        pltpu.sync_copy(
            erows.at[0, pl.ds(0, rem_rows)],
            aggsh.at[pl.ds(s * STRIPE + (STRIPE // BLK) * BLK, rem_rows)])
    plsc.subcore_barrier()

    def issue_src(r, slot):
        pltpu.async_copy(src_hbm.at[w, r], srcv.at[slot], si[slot])

    def wait_src(r, slot):
        pltpu.make_async_copy(src_hbm.at[w, r], srcv.at[slot],
                              si[slot]).wait()

    def issue_dst(r, slot):
        pltpu.async_copy(dst_hbm.at[w, r], dstv.at[slot], sj[slot])

    def wait_dst(r, slot):
        pltpu.make_async_copy(dst_hbm.at[w, r], dstv.at[slot],
                              sj[slot]).wait()

    def issue_gather(r, slot):
        pltpu.async_copy(h_hbm.at[srcv.at[slot]], hrows.at[slot], sg[slot])

    def wait_gather(r, slot):
        pltpu.make_async_copy(h_hbm.at[srcv.at[slot]], hrows.at[slot],
                              sg[slot]).wait()

    def issue_e(r, slot):
        pltpu.async_copy(e_hbm.at[w * RW + r], erows.at[slot], se[slot])

    def wait_e(r, slot):
        pltpu.make_async_copy(e_hbm.at[w * RW + r], erows.at[slot],
                              se[slot]).wait()

    def issue_scatter(r, slot):
        pltpu.async_copy(erows.at[slot], aggsh.at[dstv.at[slot]], ss[slot],
                         add=True)

    def wait_scatter(r, slot):
        pltpu.make_async_copy(erows.at[slot], aggsh.at[dstv.at[slot]],
                              ss[slot]).wait()

    issue_src(0, 0)
    issue_src(1, 1)
    issue_src(2, 2)
    issue_dst(0, 0)
    wait_src(0, 0)
    issue_gather(0, 0)
    issue_e(0, 0)
    wait_src(1, 1)
    issue_gather(1, 1)

    def outer(it, _):
        r0 = it * 3
        for b in range(3):
            r = r0 + b
            s1 = (b + 1) % 3
            s2 = (b + 2) % 3

            @pl.when(r >= 2)
            def _():
                wait_scatter(r - 2, s1)

            @pl.when(r + 2 < RW)
            def _():
                wait_src(r + 2, s2)
                issue_gather(r + 2, s2)

            @pl.when(r + 1 < RW)
            def _():
                issue_e(r + 1, s1)
                issue_dst(r + 1, s1)

            wait_gather(r, b)
            wait_e(r, b)

            @pl.when(r + 3 < RW)
            def _():
                issue_src(r + 3, b)

            def msg_row(i, _):
                for j in range(HID // 16):
                    sl = pl.ds(j * 16, 16)
                    erows[b, i, sl] = jnp.maximum(
                        hrows[b, i, sl] + erows[b, i, sl], 0.0)
                return 0
            lax.fori_loop(0, BLK, msg_row, 0)

            wait_dst(r, b)
            issue_scatter(r, b)
        return 0

    lax.fori_loop(0, RW // 3, outer, 0)
    wait_scatter(RW - 2, (RW - 2) % 3)
    wait_scatter(RW - 1, (RW - 1) % 3)

    plsc.subcore_barrier()
    pltpu.sync_copy(aggsh.at[pl.ds(s * STRIPE, STRIPE)],
                    out_hbm.at[c, pl.ds(s * STRIPE, STRIPE)])


_sc_agg = functools.partial(
    pl.kernel,
    out_type=jax.ShapeDtypeStruct((NC, NPAD, HID), jnp.float32),
    mesh=plsc.VectorSubcoreMesh(core_axis_name="c", subcore_axis_name="s"),
    scratch_types=[
        pltpu.VMEM((3, BLK), jnp.int32),
        pltpu.VMEM((3, BLK), jnp.int32),
        pltpu.VMEM((3, BLK, HID), jnp.float32),
        pltpu.VMEM((3, BLK, HID), jnp.float32),
        pltpu.VMEM_SHARED((NPAD, HID), jnp.float32),
    ] + [pltpu.SemaphoreType.DMA] * 15,
)(_sc_agg_body)


def kernel(x, edge_attr, edge_index, batch,
           W_node, b_node, W_edge, b_edge, W1, b1, W2, b2):
    src = edge_index[0]
    dst = edge_index[1]
    pad = EPAD - E
    src_p = jnp.concatenate([src, jnp.zeros((pad,), jnp.int32)])
    src_p = src_p.reshape(NW, RW, BLK)
    dst_p = jnp.concatenate([dst, jnp.full((pad,), N, jnp.int32)])
    dst_p = dst_p.reshape(NW, RW, BLK)
    batch2d = batch.reshape(N, 1)

    h = _proj_h(x, W_node, b_node.reshape(1, HID))
    e = _proj_e(edge_attr, W_edge, b_edge.reshape(1, HID))

    for i in range(DEPTH):
        agg = _sc_agg(h, e, src_p, dst_p)
        if i < DEPTH - 1:
            h = _mlp(h, agg, W1[i], b1[i].reshape(1, HID),
                     W2[i], b2[i].reshape(1, HID))
        else:
            out = _mlp_pool(h, agg, W1[i], b1[i].reshape(1, HID),
                            W2[i], b2[i].reshape(1, HID), batch2d)
    return out
